# Initial kernel scaffold; baseline (speedup 1.0000x reference)
import jax
import jax.numpy as jnp
from jax.experimental import pallas as pl


def kernel(x):
    raise NotImplementedError("write your pallas kernel here")


# speedup vs baseline: 6.6591x; 6.6591x over previous
import dataclasses
import functools

import jax
import jax.numpy as jnp
from jax import lax
from jax.experimental import pallas as pl
from jax.experimental.pallas import tpu as pltpu
from jax.experimental.pallas import tpu_sc as plsc

ROWS, COLS = 128, 32768
TOPK = 2048
LANES = 16
NTILES = 32
ROWS_PER_TILE = ROWS // NTILES
NV = COLS // LANES

DB1, DB2, DB3 = 11, 10, 10
NB1, NB2, NB3 = 1 << DB1, 1 << DB2, 1 << DB3
HIST_WORDS = NB1 * LANES


def _scan_hist(hist_ref, nbins, kth):
    ngroups = nbins // LANES

    def phase_a(g, carry):
        cnt, gsel, cnt_before, found = carry
        gi = ngroups - 1 - g
        acc = jnp.zeros((LANES,), jnp.int32)
        for j in range(LANES):
            acc = acc + hist_ref[pl.ds((gi * LANES + j) * LANES, LANES)]
        s = jnp.sum(acc)
        cross = jnp.logical_and(jnp.logical_not(found), cnt + s >= kth)
        gsel = jnp.where(cross, gi, gsel)
        cnt_before = jnp.where(cross, cnt, cnt_before)
        found = jnp.logical_or(found, cross)
        cnt = jnp.where(found, cnt, cnt + s)
        return cnt, gsel, cnt_before, found

    zero = jnp.int32(0)
    _, gsel, cnt_g, found_g = lax.fori_loop(
        0, ngroups, phase_a, (zero, zero, zero, False))

    def phase_b(j, carry):
        cnt, bsel, cnt_before, found = carry
        bi = gsel * LANES + (LANES - 1 - j)
        s = jnp.sum(hist_ref[pl.ds(bi * LANES, LANES)])
        cross = jnp.logical_and(jnp.logical_not(found), cnt + s >= kth)
        bsel = jnp.where(cross, bi, bsel)
        cnt_before = jnp.where(cross, cnt, cnt_before)
        found = jnp.logical_or(found, cross)
        cnt = jnp.where(found, cnt, cnt + s)
        return cnt, bsel, cnt_before, found

    _, bsel, cnt_b, found_b = lax.fori_loop(
        0, LANES, phase_b, (cnt_g, zero, zero, False))

    k_rem = kth - cnt_b
    return bsel, k_rem, jnp.logical_and(found_g, found_b)


def _process_row(row_ref, hist_ref):
    lanes = lax.iota(jnp.int32, LANES)
    ones = jnp.ones((LANES,), jnp.int32)
    zeros = jnp.zeros((LANES,), jnp.int32)

    @pl.loop(0, HIST_WORDS // LANES)
    def _(i):
        hist_ref[pl.ds(i * LANES, LANES)] = zeros

    @pl.loop(0, NV)
    def _(i):
        xv = row_ref[pl.ds(i * LANES, LANES)]
        key = plsc.bitcast(jnp.maximum(xv, 0.0), jnp.int32)
        m = xv > 0.0
        slot = (((key >> 20) & (NB1 - 1)) << 4) | lanes
        plsc.addupdate_scatter(hist_ref, [slot], ones, mask=m)

    b1, k1, f1 = _scan_hist(hist_ref, NB1, jnp.int32(TOPK))

    @pl.loop(0, NB2)
    def _(i):
        hist_ref[pl.ds(i * LANES, LANES)] = zeros

    @pl.loop(0, NV)
    def _(i):
        xv = row_ref[pl.ds(i * LANES, LANES)]
        key = plsc.bitcast(jnp.maximum(xv, 0.0), jnp.int32)
        m = jnp.logical_and(xv > 0.0, ((key >> 20) & (NB1 - 1)) == b1)
        slot = (((key >> 10) & (NB2 - 1)) << 4) | lanes
        plsc.addupdate_scatter(hist_ref, [slot], ones, mask=m)

    b2, k2, f2 = _scan_hist(hist_ref, NB2, k1)

    @pl.loop(0, NB3)
    def _(i):
        hist_ref[pl.ds(i * LANES, LANES)] = zeros

    pref21 = (b1 << 10) | b2

    @pl.loop(0, NV)
    def _(i):
        xv = row_ref[pl.ds(i * LANES, LANES)]
        key = plsc.bitcast(jnp.maximum(xv, 0.0), jnp.int32)
        m = jnp.logical_and(
            xv > 0.0, ((key >> 10) & ((1 << 21) - 1)) == pref21)
        slot = ((key & (NB3 - 1)) << 4) | lanes
        plsc.addupdate_scatter(hist_ref, [slot], ones, mask=m)

    b3, _, f3 = _scan_hist(hist_ref, NB3, k2)

    thresh = (b1 << 20) | (b2 << 10) | b3
    ok = jnp.logical_and(f1, jnp.logical_and(f2, f3))
    thresh = jnp.where(ok, thresh, 0)
    tvec = jnp.full((LANES,), thresh, jnp.int32)

    @pl.loop(0, NV)
    def _(i):
        xv = row_ref[pl.ds(i * LANES, LANES)]
        xr = jnp.maximum(xv, 0.0)
        key = plsc.bitcast(xr, jnp.int32)
        row_ref[pl.ds(i * LANES, LANES)] = jnp.where(key >= tvec, xr, 0.0)


def kernel(x):
    mesh = plsc.VectorSubcoreMesh(core_axis_name="c", subcore_axis_name="s")
    cp = pltpu.CompilerParams()
    if "needs_layout_passes" in pltpu.CompilerParams.__dataclass_fields__:
        cp = dataclasses.replace(cp, needs_layout_passes=False)

    @functools.partial(
        pl.kernel,
        out_type=jax.ShapeDtypeStruct((ROWS, COLS), jnp.float32),
        mesh=mesh,
        compiler_params=cp,
        scratch_types=[
            pltpu.VMEM((COLS,), jnp.float32),
            pltpu.VMEM((HIST_WORDS,), jnp.int32),
        ],
    )
    def run(x_hbm, out_hbm, row_ref, hist_ref):
        wid = lax.axis_index("s") * 2 + lax.axis_index("c")

        @pl.loop(0, ROWS_PER_TILE)
        def _(r):
            row = wid * ROWS_PER_TILE + r
            pltpu.sync_copy(x_hbm.at[row], row_ref)
            _process_row(row_ref, hist_ref)
            pltpu.sync_copy(row_ref, out_hbm.at[row])

    return run(x)


# speedup vs baseline: 9.7231x; 1.4601x over previous
import dataclasses
import functools

import jax
import jax.numpy as jnp
from jax import lax
from jax.experimental import pallas as pl
from jax.experimental.pallas import tpu as pltpu
from jax.experimental.pallas import tpu_sc as plsc

ROWS, COLS = 128, 32768
TOPK = 2048
LANES = 16
NTILES = 32
ROWS_PER_TILE = ROWS // NTILES
NV = COLS // LANES

DB1, DB2, DB3 = 11, 10, 10
NB1, NB2, NB3 = 1 << DB1, 1 << DB2, 1 << DB3
HIST_WORDS = NB1 * LANES


def _scan_hist(hist_ref, nbins, kth):
    ngroups = nbins // LANES

    def phase_a(g, carry):
        cnt, gsel, cnt_before, found = carry
        gi = ngroups - 1 - g
        acc = jnp.zeros((LANES,), jnp.int32)
        for j in range(LANES):
            acc = acc + hist_ref[pl.ds((gi * LANES + j) * LANES, LANES)]
        s = jnp.sum(acc)
        cross = jnp.logical_and(jnp.logical_not(found), cnt + s >= kth)
        gsel = jnp.where(cross, gi, gsel)
        cnt_before = jnp.where(cross, cnt, cnt_before)
        found = jnp.logical_or(found, cross)
        cnt = jnp.where(found, cnt, cnt + s)
        return cnt, gsel, cnt_before, found

    zero = jnp.int32(0)
    _, gsel, cnt_g, found_g = lax.fori_loop(
        0, ngroups, phase_a, (zero, zero, zero, False))

    def phase_b(j, carry):
        cnt, bsel, cnt_before, found = carry
        bi = gsel * LANES + (LANES - 1 - j)
        s = jnp.sum(hist_ref[pl.ds(bi * LANES, LANES)])
        cross = jnp.logical_and(jnp.logical_not(found), cnt + s >= kth)
        bsel = jnp.where(cross, bi, bsel)
        cnt_before = jnp.where(cross, cnt, cnt_before)
        found = jnp.logical_or(found, cross)
        cnt = jnp.where(found, cnt, cnt + s)
        return cnt, bsel, cnt_before, found

    _, bsel, cnt_b, found_b = lax.fori_loop(
        0, LANES, phase_b, (cnt_g, zero, zero, False))

    k_rem = kth - cnt_b
    return bsel, k_rem, jnp.logical_and(found_g, found_b)


UNROLL = 8


def _process_row(row_ref, hist_ref):
    lanes = lax.iota(jnp.int32, LANES)
    ones = jnp.ones((LANES,), jnp.int32)
    zeros = jnp.zeros((LANES,), jnp.int32)

    def clear(nwords):
        @pl.loop(0, nwords // LANES, step=UNROLL)
        def _(i):
            for u in range(UNROLL):
                hist_ref[pl.ds((i + u) * LANES, LANES)] = zeros

    clear(HIST_WORDS)

    @pl.loop(0, NV, step=UNROLL)
    def _(i):
        for u in range(UNROLL):
            xv = row_ref[pl.ds((i + u) * LANES, LANES)]
            key = plsc.bitcast(jnp.maximum(xv, 0.0), jnp.int32)
            m = xv > 0.0
            slot = (((key >> 20) & (NB1 - 1)) << 4) | lanes
            plsc.addupdate_scatter(hist_ref, [slot], ones, mask=m)

    b1, k1, f1 = _scan_hist(hist_ref, NB1, jnp.int32(TOPK))

    clear(NB2 * LANES)

    @pl.loop(0, NV, step=UNROLL)
    def _(i):
        for u in range(UNROLL):
            xv = row_ref[pl.ds((i + u) * LANES, LANES)]
            key = plsc.bitcast(jnp.maximum(xv, 0.0), jnp.int32)
            m = jnp.logical_and(xv > 0.0, ((key >> 20) & (NB1 - 1)) == b1)
            slot = (((key >> 10) & (NB2 - 1)) << 4) | lanes
            plsc.addupdate_scatter(hist_ref, [slot], ones, mask=m)

    b2, k2, f2 = _scan_hist(hist_ref, NB2, k1)

    clear(NB3 * LANES)

    pref21 = (b1 << 10) | b2

    @pl.loop(0, NV, step=UNROLL)
    def _(i):
        for u in range(UNROLL):
            xv = row_ref[pl.ds((i + u) * LANES, LANES)]
            key = plsc.bitcast(jnp.maximum(xv, 0.0), jnp.int32)
            m = jnp.logical_and(
                xv > 0.0, ((key >> 10) & ((1 << 21) - 1)) == pref21)
            slot = ((key & (NB3 - 1)) << 4) | lanes
            plsc.addupdate_scatter(hist_ref, [slot], ones, mask=m)

    b3, _, f3 = _scan_hist(hist_ref, NB3, k2)

    thresh = (b1 << 20) | (b2 << 10) | b3
    ok = jnp.logical_and(f1, jnp.logical_and(f2, f3))
    thresh = jnp.where(ok, thresh, 0)
    tvec = jnp.full((LANES,), thresh, jnp.int32)

    @pl.loop(0, NV, step=UNROLL)
    def _(i):
        for u in range(UNROLL):
            xv = row_ref[pl.ds((i + u) * LANES, LANES)]
            xr = jnp.maximum(xv, 0.0)
            key = plsc.bitcast(xr, jnp.int32)
            row_ref[pl.ds((i + u) * LANES, LANES)] = jnp.where(
                key >= tvec, xr, 0.0)


def kernel(x):
    mesh = plsc.VectorSubcoreMesh(core_axis_name="c", subcore_axis_name="s")
    cp = pltpu.CompilerParams()
    if "needs_layout_passes" in pltpu.CompilerParams.__dataclass_fields__:
        cp = dataclasses.replace(cp, needs_layout_passes=False)

    @functools.partial(
        pl.kernel,
        out_type=jax.ShapeDtypeStruct((ROWS, COLS), jnp.float32),
        mesh=mesh,
        compiler_params=cp,
        scratch_types=[
            pltpu.VMEM((COLS,), jnp.float32),
            pltpu.VMEM((HIST_WORDS,), jnp.int32),
        ],
    )
    def run(x_hbm, out_hbm, row_ref, hist_ref):
        wid = lax.axis_index("s") * 2 + lax.axis_index("c")

        @pl.loop(0, ROWS_PER_TILE)
        def _(r):
            row = wid * ROWS_PER_TILE + r
            pltpu.sync_copy(x_hbm.at[row], row_ref)
            _process_row(row_ref, hist_ref)
            pltpu.sync_copy(row_ref, out_hbm.at[row])

    return run(x)


# speedup vs baseline: 12.2419x; 1.2591x over previous
import dataclasses
import functools

import jax
import jax.numpy as jnp
from jax import lax
from jax.experimental import pallas as pl
from jax.experimental.pallas import tpu as pltpu
from jax.experimental.pallas import tpu_sc as plsc

ROWS, COLS = 128, 32768
TOPK = 2048
LANES = 16
NTILES = 32
ROWS_PER_TILE = ROWS // NTILES
NV = COLS // LANES

DB1, DB2, DB3 = 11, 10, 10
NB1, NB2, NB3 = 1 << DB1, 1 << DB2, 1 << DB3
HIST_WORDS = NB1 * LANES


def _scan_hist(hist_ref, nbins, kth):
    ngroups = nbins // LANES

    def phase_a(g, carry):
        cnt, gsel, cnt_before, found = carry
        gi = ngroups - 1 - g
        acc = jnp.zeros((LANES,), jnp.int32)
        for j in range(LANES):
            acc = acc + hist_ref[pl.ds((gi * LANES + j) * LANES, LANES)]
        s = jnp.sum(acc)
        cross = jnp.logical_and(jnp.logical_not(found), cnt + s >= kth)
        gsel = jnp.where(cross, gi, gsel)
        cnt_before = jnp.where(cross, cnt, cnt_before)
        found = jnp.logical_or(found, cross)
        cnt = jnp.where(found, cnt, cnt + s)
        return cnt, gsel, cnt_before, found

    zero = jnp.int32(0)
    _, gsel, cnt_g, found_g = lax.fori_loop(
        0, ngroups, phase_a, (zero, zero, zero, False))

    def phase_b(j, carry):
        cnt, bsel, cnt_before, found = carry
        bi = gsel * LANES + (LANES - 1 - j)
        s = jnp.sum(hist_ref[pl.ds(bi * LANES, LANES)])
        cross = jnp.logical_and(jnp.logical_not(found), cnt + s >= kth)
        bsel = jnp.where(cross, bi, bsel)
        cnt_before = jnp.where(cross, cnt, cnt_before)
        found = jnp.logical_or(found, cross)
        cnt = jnp.where(found, cnt, cnt + s)
        return cnt, bsel, cnt_before, found

    _, bsel, cnt_b, found_b = lax.fori_loop(
        0, LANES, phase_b, (cnt_g, zero, zero, False))

    k_rem = kth - cnt_b
    return bsel, k_rem, jnp.logical_and(found_g, found_b)


UNROLL = 8


def _process_row(row_ref, hist_ref, buf_ref):
    lanes = lax.iota(jnp.int32, LANES)
    ones = jnp.ones((LANES,), jnp.int32)
    zeros = jnp.zeros((LANES,), jnp.int32)

    def clear(nwords):
        @pl.loop(0, nwords // LANES, step=UNROLL)
        def _(i):
            for u in range(UNROLL):
                hist_ref[pl.ds((i + u) * LANES, LANES)] = zeros

    clear(HIST_WORDS)

    @pl.loop(0, NV, step=UNROLL)
    def _(i):
        for u in range(UNROLL):
            xv = row_ref[pl.ds((i + u) * LANES, LANES)]
            key = plsc.bitcast(jnp.maximum(xv, 0.0), jnp.int32)
            m = xv > 0.0
            slot = (((key >> 20) & (NB1 - 1)) << 4) | lanes
            plsc.addupdate_scatter(hist_ref, [slot], ones, mask=m)

    b1, k1, f1 = _scan_hist(hist_ref, NB1, jnp.int32(TOPK))

    def compact_step(i, off):
        xv = row_ref[pl.ds(i * LANES, LANES)]
        key = plsc.bitcast(jnp.maximum(xv, 0.0), jnp.int32)
        m = jnp.logical_and(xv > 0.0, ((key >> 20) & (NB1 - 1)) == b1)
        plsc.store_compressed(buf_ref.at[pl.ds(off, LANES)], key, mask=m)
        return off + jnp.max(plsc.all_reduce_population_count(m))

    n_cand = lax.fori_loop(0, NV, compact_step, jnp.int32(0))
    buf_ref[pl.ds(n_cand, LANES)] = jnp.zeros((LANES,), jnp.int32)
    nvb = (n_cand + LANES - 1) >> 4

    low_mask = (1 << 20) - 1

    def bit_step(bi, prefix):
        trial = prefix | (1 << (19 - bi))

        def cnt_step(j, acc):
            kv = buf_ref[pl.ds(j * LANES, LANES)]
            lo = kv & low_mask
            return acc + jnp.where(lo >= trial, 1, 0)

        accv = lax.fori_loop(0, nvb, cnt_step,
                             jnp.zeros((LANES,), jnp.int32))
        return jnp.where(jnp.sum(accv) >= k1, trial, prefix)

    low20 = lax.fori_loop(0, 20, bit_step, jnp.int32(0))

    thresh = (b1 << 20) | low20
    thresh = jnp.where(f1, thresh, 0)
    tvec = jnp.full((LANES,), thresh, jnp.int32)

    @pl.loop(0, NV, step=UNROLL)
    def _(i):
        for u in range(UNROLL):
            xv = row_ref[pl.ds((i + u) * LANES, LANES)]
            xr = jnp.maximum(xv, 0.0)
            key = plsc.bitcast(xr, jnp.int32)
            row_ref[pl.ds((i + u) * LANES, LANES)] = jnp.where(
                key >= tvec, xr, 0.0)


def kernel(x):
    mesh = plsc.VectorSubcoreMesh(core_axis_name="c", subcore_axis_name="s")
    cp = pltpu.CompilerParams()
    if "needs_layout_passes" in pltpu.CompilerParams.__dataclass_fields__:
        cp = dataclasses.replace(cp, needs_layout_passes=False)

    @functools.partial(
        pl.kernel,
        out_type=jax.ShapeDtypeStruct((ROWS, COLS), jnp.float32),
        mesh=mesh,
        compiler_params=cp,
        scratch_types=[
            pltpu.VMEM((COLS,), jnp.float32),
            pltpu.VMEM((HIST_WORDS,), jnp.int32),
            pltpu.VMEM((COLS + LANES,), jnp.int32),
        ],
    )
    def run(x_hbm, out_hbm, row_ref, hist_ref, buf_ref):
        wid = lax.axis_index("s") * 2 + lax.axis_index("c")

        @pl.loop(0, ROWS_PER_TILE)
        def _(r):
            row = wid * ROWS_PER_TILE + r
            pltpu.sync_copy(x_hbm.at[row], row_ref)
            _process_row(row_ref, hist_ref, buf_ref)
            pltpu.sync_copy(row_ref, out_hbm.at[row])

    return run(x)
